# baseline (speedup 1.0000x reference)
import functools

import jax
import jax.numpy as jnp
from jax import lax
from jax.experimental import pallas as pl
from jax.experimental.pallas import tpu as pltpu
from jax.experimental.pallas import tpu_sc as plsc

BATCH = 16384
DIM = 32
NUM_CORES = 2
NUM_SUBCORES = 16
NW = NUM_CORES * NUM_SUBCORES
B_PER_W = BATCH // NW
GROUP = 16
NGROUP = B_PER_W // GROUP


CHUNK = 128
NCHUNK = B_PER_W // CHUNK


def _gather_body(x_hbm, tt_hbm, out_hbm, idx_v, rows_v, sem):
    wid = lax.axis_index("s") * NUM_CORES + lax.axis_index("c")
    base = wid * B_PER_W
    pltpu.sync_copy(x_hbm.at[wid], idx_v)

    def per_feature(f, carry):
        copies = [
            pltpu.async_copy(
                tt_hbm.at[f].at[idx_v.at[j]],
                rows_v.at[f, pl.ds(j * CHUNK, CHUNK)],
                sem,
            )
            for j in range(NCHUNK)
        ]
        for c in copies:
            c.wait()
        return carry

    lax.fori_loop(0, DIM, per_feature, 0)
    pltpu.sync_copy(rows_v, out_hbm.at[:, pl.ds(base, B_PER_W)])


@jax.jit
def _embedding_gather(x3, table_t):
    mesh = plsc.VectorSubcoreMesh(core_axis_name="c", subcore_axis_name="s")
    k = functools.partial(
        pl.kernel,
        mesh=mesh,
        out_type=jax.ShapeDtypeStruct((DIM, BATCH), jnp.float32),
        scratch_types=[
            pltpu.VMEM((NCHUNK, CHUNK), jnp.int32),
            pltpu.VMEM((DIM, B_PER_W), jnp.float32),
            pltpu.SemaphoreType.DMA,
        ],
        compiler_params=pltpu.CompilerParams(use_tc_tiling_on_sc=False),
    )(_gather_body)
    return k(x3, table_t)


def kernel(x, table):
    x3 = x.astype(jnp.int32).reshape(NW, NCHUNK, CHUNK)
    out_t = _embedding_gather(x3, table.T)
    return out_t.T


# speedup vs baseline: 15.5408x; 15.5408x over previous
import functools

import jax
import jax.numpy as jnp
from jax import lax
from jax.experimental import pallas as pl
from jax.experimental.pallas import tpu as pltpu
from jax.experimental.pallas import tpu_sc as plsc

BATCH = 16384
DIM = 32
NUM_CORES = 2
NUM_SUBCORES = 16
NW = NUM_CORES * NUM_SUBCORES
B_PER_W = BATCH // NW
NBUF = 4


def _gather_body(x_hbm, tt_hbm, out_hbm, idx_v, blocks_v, rows_v, sems):
    wid = lax.axis_index("s") * NUM_CORES + lax.axis_index("c")
    pltpu.sync_copy(x_hbm.at[wid], idx_v.at[pl.ds(0, B_PER_W)])

    fvec0 = lax.iota(jnp.int32, 16)
    fvec1 = fvec0 + 16

    def fetch(i, slot):
        idx = idx_v[pl.ds(i, 16)][0]
        col = pl.multiple_of((idx >> 7) * 128, 128)
        return pltpu.async_copy(
            tt_hbm.at[:, pl.ds(col, 128)], blocks_v.at[slot], sems.at[slot]
        )

    for b in range(NBUF):
        fetch(b, b)

    def per_index(i, carry):
        slot = lax.rem(i, NBUF)
        pltpu.make_async_copy(
            tt_hbm.at[:, pl.ds(0, 128)], blocks_v.at[slot], sems.at[slot]
        ).wait()
        lane = jnp.broadcast_to(idx_v[pl.ds(i, 16)][0] & 127, (16,))
        col0 = plsc.load_gather(blocks_v.at[slot], [fvec0, lane])
        col1 = plsc.load_gather(blocks_v.at[slot], [fvec1, lane])
        rows_v[i, pl.ds(0, 16)] = col0
        rows_v[i, pl.ds(16, 16)] = col1

        @pl.when(i + NBUF < B_PER_W)
        def _():
            fetch(i + NBUF, slot)

        return carry

    lax.fori_loop(0, B_PER_W, per_index, 0)
    pltpu.sync_copy(rows_v, out_hbm.at[wid])


@jax.jit
def _embedding_gather(x2, table_t):
    mesh = plsc.VectorSubcoreMesh(core_axis_name="c", subcore_axis_name="s")
    k = functools.partial(
        pl.kernel,
        mesh=mesh,
        out_type=jax.ShapeDtypeStruct((NW, B_PER_W, DIM), jnp.float32),
        scratch_types=[
            pltpu.VMEM((B_PER_W + 16,), jnp.int32),
            pltpu.VMEM((NBUF, DIM, 128), jnp.float32),
            pltpu.VMEM((B_PER_W, DIM), jnp.float32),
            pltpu.SemaphoreType.DMA((NBUF,)),
        ],
        compiler_params=pltpu.CompilerParams(needs_layout_passes=False),
    )(_gather_body)
    return k(x2, table_t)


def kernel(x, table):
    x2 = x.astype(jnp.int32).reshape(NW, B_PER_W)
    out = _embedding_gather(x2, table.T)
    return out.reshape(BATCH, DIM)


# speedup vs baseline: 19.9242x; 1.2821x over previous
import functools

import jax
import jax.numpy as jnp
from jax import lax
from jax.experimental import pallas as pl
from jax.experimental.pallas import tpu as pltpu
from jax.experimental.pallas import tpu_sc as plsc

BATCH = 16384
DIM = 32
NUM_CORES = 2
NUM_SUBCORES = 16
NW = NUM_CORES * NUM_SUBCORES
B_PER_W = BATCH // NW
NBUF = 8


def _gather_body(x_hbm, tt_hbm, out_hbm, idx_v, blocks_v, rows_v, sems):
    wid = lax.axis_index("s") * NUM_CORES + lax.axis_index("c")
    pltpu.sync_copy(x_hbm.at[wid], idx_v.at[pl.ds(0, B_PER_W)])

    fvec0 = lax.iota(jnp.int32, 16)
    fvec1 = fvec0 + 16

    def fetch(i, slot):
        idx = idx_v[pl.ds(i, 16)][0]
        col = pl.multiple_of((idx >> 7) * 128, 128)
        return pltpu.async_copy(
            tt_hbm.at[:, pl.ds(col, 128)], blocks_v.at[slot], sems.at[slot]
        )

    for b in range(NBUF):
        fetch(b, b)

    def per_index(i, carry):
        slot = lax.rem(i, NBUF)
        pltpu.make_async_copy(
            tt_hbm.at[:, pl.ds(0, 128)], blocks_v.at[slot], sems.at[slot]
        ).wait()
        lane = jnp.broadcast_to(idx_v[pl.ds(i, 16)][0] & 127, (16,))
        col0 = plsc.load_gather(blocks_v.at[slot], [fvec0, lane])
        col1 = plsc.load_gather(blocks_v.at[slot], [fvec1, lane])
        rows_v[i, pl.ds(0, 16)] = col0
        rows_v[i, pl.ds(16, 16)] = col1

        @pl.when(i + NBUF < B_PER_W)
        def _():
            fetch(i + NBUF, slot)

        return carry

    lax.fori_loop(0, B_PER_W, per_index, 0)
    pltpu.sync_copy(rows_v, out_hbm.at[wid])


@jax.jit
def _embedding_gather(x2, table_t):
    mesh = plsc.VectorSubcoreMesh(core_axis_name="c", subcore_axis_name="s")
    k = functools.partial(
        pl.kernel,
        mesh=mesh,
        out_type=jax.ShapeDtypeStruct((NW, B_PER_W, DIM), jnp.float32),
        scratch_types=[
            pltpu.VMEM((B_PER_W + 16,), jnp.int32),
            pltpu.VMEM((NBUF, DIM, 128), jnp.float32),
            pltpu.VMEM((B_PER_W, DIM), jnp.float32),
            pltpu.SemaphoreType.DMA((NBUF,)),
        ],
        compiler_params=pltpu.CompilerParams(needs_layout_passes=False),
    )(_gather_body)
    return k(x2, table_t)


def kernel(x, table):
    x2 = x.astype(jnp.int32).reshape(NW, B_PER_W)
    out = _embedding_gather(x2, table.T)
    return out.reshape(BATCH, DIM)
